# baseline (speedup 1.0000x reference)
import functools

import jax
import jax.numpy as jnp
from jax import lax
from jax.experimental import pallas as pl
from jax.experimental.pallas import tpu as pltpu

_NEG = -1e30
_RT = 256


def _body(band_ref, x_ref, feat_ref, val_ref, alpha_ref, beta_ref, gamma_ref,
          l2r_ref, params_ref, out_ref, counts_ref, *, K):
  j = pl.program_id(1)
  nj = pl.num_programs(1)
  Bb, F = x_ref.shape
  Lt = feat_ref.shape[-1]
  R = counts_ref.shape[1]
  W = params_ref.shape[1]

  @pl.when(j == 0)
  def _init():
    counts_ref[...] = jnp.zeros_like(counts_ref)

  x = x_ref[...]
  feat = feat_ref[...].reshape(1, Lt)
  fo = jnp.where(
      lax.broadcasted_iota(jnp.int32, (F, Lt), 0) == jnp.broadcast_to(feat, (F, Lt)),
      1.0, 0.0).astype(jnp.bfloat16)
  a = x.astype(jnp.bfloat16)
  r1 = x - a.astype(jnp.float32)
  b = r1.astype(jnp.bfloat16)
  c = (r1 - b.astype(jnp.float32)).astype(jnp.bfloat16)
  abc = jnp.concatenate([a, b, c], axis=1)
  fo3 = jnp.concatenate([fo, fo, fo], axis=0)
  vals = jnp.dot(abc, fo3, preferred_element_type=jnp.float32)

  v = jnp.broadcast_to(val_ref[...].reshape(1, Lt), (Bb, Lt))
  alpha = jnp.broadcast_to(alpha_ref[...].reshape(1, Lt), (Bb, Lt))
  beta = jnp.broadcast_to(beta_ref[...].reshape(1, Lt), (Bb, Lt))
  gamma = jnp.broadcast_to(gamma_ref[...].reshape(1, Lt), (Bb, Lt))
  d = vals - v
  g = alpha * d + beta * jnp.abs(d) + gamma
  miss = jnp.where(g > 0, 0.0, 1.0).astype(jnp.bfloat16)

  lo = band_ref[0, j]
  hi = band_ref[1, j]
  l2r_col = l2r_ref[...]
  for rt in range(R // _RT):
    r0 = rt * _RT

    @pl.when(jnp.logical_and(lo < r0 + _RT, hi >= r0))
    def _acc(r0=r0):
      ind = jnp.where(
          jnp.broadcast_to(l2r_col, (Lt, _RT))
          == r0 + lax.broadcasted_iota(jnp.int32, (Lt, _RT), 1),
          1.0, 0.0).astype(jnp.bfloat16)
      counts_ref[:, r0:r0 + _RT] += jnp.dot(
          miss, ind, preferred_element_type=jnp.float32)

  @pl.when(j == nj - 1)
  def _finish():
    params = params_ref[...]
    pm = jnp.max(params, axis=1, keepdims=True)
    e = jnp.exp(params - pm)
    m = e / jnp.sum(e, axis=1, keepdims=True)
    kio = lax.broadcasted_iota(jnp.int32, (R, W), 1)
    m_om = jnp.sum(jnp.where(kio == K, m, 0.0), axis=1, keepdims=True)
    logcomm = jnp.log(jnp.maximum(m + m_om, 1e-12))
    logom = jnp.broadcast_to(jnp.log(jnp.maximum(m_om, 1e-12)), (R, W))
    P = jnp.where(kio < K, logcomm, jnp.where(kio == K, logom, 0.0))

    fired = jnp.where(counts_ref[...] == 0.0, 1.0, 0.0)
    logq = jnp.dot(fired.astype(jnp.bfloat16), P.astype(jnp.bfloat16),
                   preferred_element_type=jnp.float32)
    it = lax.broadcasted_iota(jnp.int32, (Bb, W), 1)
    lq_c = jnp.where(it < K, logq, _NEG)
    lq_om = jnp.sum(jnp.where(it == K, logq, 0.0), axis=1, keepdims=True)
    mx = jnp.maximum(jnp.max(lq_c, axis=1, keepdims=True), lq_om)
    q_c = jnp.exp(lq_c - mx)
    q_om = jnp.exp(lq_om - mx)
    norm = jnp.maximum(jnp.sum(q_c, axis=1, keepdims=True) - (K - 1) * q_om,
                       1e-12)
    betp = (q_c - q_om) / norm + (q_om / norm) * (1.0 / K)
    betp = jnp.where(it < K, betp, 0.0)
    betp = betp / jnp.maximum(jnp.sum(betp, axis=1, keepdims=True), 1e-12)
    out_ref[...] = betp


def kernel(X, lit_value, rule_mass_params, lit_feat_idx, lit_op_code, lit2rule):
  B, F = X.shape
  L = lit_value.shape[0]
  R, K1 = rule_mass_params.shape
  K = K1 - 1
  W = 16
  Bb = min(2048, B)
  Lt = min(2048, L)
  ni, nj = B // Bb, L // Lt

  l2r_i = lit2rule.astype(jnp.int32)
  l2r_t = l2r_i.reshape(nj, Lt)
  band = jnp.stack([l2r_t[:, 0], l2r_t[:, -1]])
  feat3 = lit_feat_idx.astype(jnp.int32).reshape(nj, 1, Lt)
  val3 = lit_value.reshape(nj, 1, Lt)
  is0 = lit_op_code == 0
  is1 = lit_op_code == 1
  alpha3 = jnp.where(is0, 0.0, jnp.where(is1, -1.0, 1.0)).reshape(nj, 1, Lt)
  beta3 = jnp.where(is0, -1.0, 0.0).reshape(nj, 1, Lt)
  gamma3 = jnp.where(is0, 1e-6, 0.0).reshape(nj, 1, Lt)
  l2r2 = l2r_i.reshape(L, 1)
  params_p = jnp.pad(rule_mass_params, ((0, 0), (0, W - K1)),
                     constant_values=_NEG)

  out = pl.pallas_call(
      functools.partial(_body, K=K),
      grid_spec=pltpu.PrefetchScalarGridSpec(
          num_scalar_prefetch=1,
          grid=(ni, nj),
          in_specs=[
              pl.BlockSpec((Bb, F), lambda i, j, *_: (i, 0)),
              pl.BlockSpec((1, 1, Lt), lambda i, j, *_: (j, 0, 0)),
              pl.BlockSpec((1, 1, Lt), lambda i, j, *_: (j, 0, 0)),
              pl.BlockSpec((1, 1, Lt), lambda i, j, *_: (j, 0, 0)),
              pl.BlockSpec((1, 1, Lt), lambda i, j, *_: (j, 0, 0)),
              pl.BlockSpec((1, 1, Lt), lambda i, j, *_: (j, 0, 0)),
              pl.BlockSpec((Lt, 1), lambda i, j, *_: (j, 0)),
              pl.BlockSpec((R, W), lambda i, j, *_: (0, 0)),
          ],
          out_specs=pl.BlockSpec((Bb, W), lambda i, j, *_: (i, 0)),
          scratch_shapes=[pltpu.VMEM((Bb, R), jnp.float32)],
      ),
      out_shape=jax.ShapeDtypeStruct((B, W), jnp.float32),
      compiler_params=pltpu.CompilerParams(
          dimension_semantics=("parallel", "arbitrary")),
  )(band, X, feat3, val3, alpha3, beta3, gamma3, l2r2, params_p)
  return out[:, :K]


# speedup vs baseline: 2.5126x; 2.5126x over previous
import functools

import jax
import jax.numpy as jnp
from jax import lax
from jax.experimental import pallas as pl
from jax.experimental.pallas import tpu as pltpu

_NEG = -1e30
_RT = 256


def _body(band_ref, x_ref, feat_ref, val_ref, alpha_ref, beta_ref, gamma_ref,
          l2r_ref, params_ref, out_ref, counts_ref, *, K):
  j = pl.program_id(1)
  nj = pl.num_programs(1)
  Bb, F = x_ref.shape
  Lt = feat_ref.shape[-1]
  R = counts_ref.shape[1]
  W = params_ref.shape[1]

  @pl.when(j == 0)
  def _init():
    counts_ref[...] = jnp.zeros_like(counts_ref)

  F3 = x_ref.shape[1]
  F = F3 // 3
  abc = x_ref[...]
  feat = feat_ref[...].reshape(1, Lt)
  fo3 = jnp.where(
      (lax.broadcasted_iota(jnp.int32, (F3, Lt), 0) & (F - 1))
      == jnp.broadcast_to(feat, (F3, Lt)),
      1.0, 0.0).astype(jnp.bfloat16)
  vals = jnp.dot(abc, fo3, preferred_element_type=jnp.float32)

  v = jnp.broadcast_to(val_ref[...].reshape(1, Lt), (Bb, Lt))
  alpha = jnp.broadcast_to(alpha_ref[...].reshape(1, Lt), (Bb, Lt))
  beta = jnp.broadcast_to(beta_ref[...].reshape(1, Lt), (Bb, Lt))
  ngamma = jnp.broadcast_to(gamma_ref[...].reshape(1, Lt), (Bb, Lt))
  d = vals - v
  g = alpha * d + beta * jnp.abs(d)
  miss = jnp.where(g <= ngamma, 1.0, 0.0).astype(jnp.bfloat16)

  lo = band_ref[0, j]
  hi = band_ref[1, j]
  l2r_col = l2r_ref[...]
  for rt in range(R // _RT):
    r0 = rt * _RT

    @pl.when(jnp.logical_and(lo < r0 + _RT, hi >= r0))
    def _acc(r0=r0):
      ind = jnp.where(
          jnp.broadcast_to(l2r_col, (Lt, _RT))
          == r0 + lax.broadcasted_iota(jnp.int32, (Lt, _RT), 1),
          1.0, 0.0).astype(jnp.bfloat16)
      counts_ref[:, r0:r0 + _RT] += jnp.dot(
          miss, ind, preferred_element_type=jnp.float32)

  @pl.when(j == nj - 1)
  def _finish():
    params = params_ref[...]
    pm = jnp.max(params, axis=1, keepdims=True)
    e = jnp.exp(params - pm)
    m = e / jnp.sum(e, axis=1, keepdims=True)
    kio = lax.broadcasted_iota(jnp.int32, (R, W), 1)
    m_om = jnp.sum(jnp.where(kio == K, m, 0.0), axis=1, keepdims=True)
    logcomm = jnp.log(jnp.maximum(m + m_om, 1e-12))
    logom = jnp.broadcast_to(jnp.log(jnp.maximum(m_om, 1e-12)), (R, W))
    P = jnp.where(kio < K, logcomm, jnp.where(kio == K, logom, 0.0))

    fired = jnp.where(counts_ref[...] == 0.0, 1.0, 0.0)
    logq = jnp.dot(fired.astype(jnp.bfloat16), P.astype(jnp.bfloat16),
                   preferred_element_type=jnp.float32)
    it = lax.broadcasted_iota(jnp.int32, (Bb, W), 1)
    lq_c = jnp.where(it < K, logq, _NEG)
    lq_om = jnp.sum(jnp.where(it == K, logq, 0.0), axis=1, keepdims=True)
    mx = jnp.maximum(jnp.max(lq_c, axis=1, keepdims=True), lq_om)
    q_c = jnp.exp(lq_c - mx)
    q_om = jnp.exp(lq_om - mx)
    norm = jnp.maximum(jnp.sum(q_c, axis=1, keepdims=True) - (K - 1) * q_om,
                       1e-12)
    betp = (q_c - q_om) / norm + (q_om / norm) * (1.0 / K)
    betp = jnp.where(it < K, betp, 0.0)
    betp = betp / jnp.maximum(jnp.sum(betp, axis=1, keepdims=True), 1e-12)
    out_ref[...] = betp


def kernel(X, lit_value, rule_mass_params, lit_feat_idx, lit_op_code, lit2rule):
  B, F = X.shape
  L = lit_value.shape[0]
  R, K1 = rule_mass_params.shape
  K = K1 - 1
  W = 16
  Bb = min(2048, B)
  Lt = min(1024, L)
  ni, nj = B // Bb, L // Lt

  l2r_i = lit2rule.astype(jnp.int32)
  l2r_t = l2r_i.reshape(nj, Lt)
  band = jnp.stack([l2r_t[:, 0], l2r_t[:, -1]])
  feat3 = lit_feat_idx.astype(jnp.int32).reshape(nj, 1, Lt)
  val3 = lit_value.reshape(nj, 1, Lt)
  is0 = lit_op_code == 0
  is1 = lit_op_code == 1
  alpha3 = jnp.where(is0, 0.0, jnp.where(is1, -1.0, 1.0)).reshape(nj, 1, Lt)
  beta3 = jnp.where(is0, -1.0, 0.0).reshape(nj, 1, Lt)
  ngamma3 = jnp.where(is0, -1e-6, 0.0).reshape(nj, 1, Lt)
  l2r2 = l2r_i.reshape(L, 1)
  a_s = X.astype(jnp.bfloat16)
  r1_s = X - a_s.astype(jnp.float32)
  b_s = r1_s.astype(jnp.bfloat16)
  c_s = (r1_s - b_s.astype(jnp.float32)).astype(jnp.bfloat16)
  abc = jnp.concatenate([a_s, b_s, c_s], axis=1)
  params_p = jnp.pad(rule_mass_params, ((0, 0), (0, W - K1)),
                     constant_values=_NEG)

  out = pl.pallas_call(
      functools.partial(_body, K=K),
      grid_spec=pltpu.PrefetchScalarGridSpec(
          num_scalar_prefetch=1,
          grid=(ni, nj),
          in_specs=[
              pl.BlockSpec((Bb, 3 * F), lambda i, j, *_: (i, 0)),
              pl.BlockSpec((1, 1, Lt), lambda i, j, *_: (j, 0, 0)),
              pl.BlockSpec((1, 1, Lt), lambda i, j, *_: (j, 0, 0)),
              pl.BlockSpec((1, 1, Lt), lambda i, j, *_: (j, 0, 0)),
              pl.BlockSpec((1, 1, Lt), lambda i, j, *_: (j, 0, 0)),
              pl.BlockSpec((1, 1, Lt), lambda i, j, *_: (j, 0, 0)),
              pl.BlockSpec((Lt, 1), lambda i, j, *_: (j, 0)),
              pl.BlockSpec((R, W), lambda i, j, *_: (0, 0)),
          ],
          out_specs=pl.BlockSpec((Bb, W), lambda i, j, *_: (i, 0)),
          scratch_shapes=[pltpu.VMEM((Bb, R), jnp.float32)],
      ),
      out_shape=jax.ShapeDtypeStruct((B, W), jnp.float32),
      compiler_params=pltpu.CompilerParams(
          dimension_semantics=("parallel", "arbitrary")),
  )(band, abc, feat3, val3, alpha3, beta3, ngamma3, l2r2, params_p)
  return out[:, :K]
